# baseline (speedup 1.0000x reference)
import functools
import numpy as np
import jax
import jax.numpy as jnp
from jax import lax
from jax.experimental import pallas as pl
from jax.experimental.pallas import tpu as pltpu
from jax.experimental.pallas import tpu_sc as plsc

MARGIN_ = 0.5
C_ = 256
K_ = 16
B_ = 64
E_ = 4096
L_ = 16
NW_ = 32
CPW_ = C_ // NW_
BIG_ = np.float32(3.0e38)


def _tc_body(preds_ref, target_ref, x_ref, d_ref, valid_ref, count_ref):
    f32 = jnp.float32
    t = target_ref[...]

    iota_cc_r = lax.broadcasted_iota(jnp.int32, (C_, C_), 0)
    iota_cc_c = lax.broadcasted_iota(jnp.int32, (C_, C_), 1)

    tf = t.astype(f32)
    h_row = jnp.sum(tf, axis=0, keepdims=True)
    ones_b1 = jnp.ones((B_, 1), f32)
    h_col = lax.dot_general(tf, ones_b1, (((0,), (0,)), ((), ())),
                            precision=lax.Precision.HIGHEST)

    before = (h_col < h_row) | ((h_col == h_row) & (iota_cc_r <= iota_cc_c))
    s_cum = jnp.sum(h_col * before.astype(f32), axis=0, keepdims=True)
    minority = (s_cum <= 0.5 * C_) & (h_row > 1.0)

    n_c = h_row
    kp = jnp.minimum(n_c - 1.0, float(K_))
    kn = jnp.minimum(float(B_) - n_c, float(K_))
    class_valid = minority & (n_c < float(B_))
    valid_ref[...] = class_valid.astype(f32)
    count_ref[...] = jnp.sum(jnp.where(class_valid, n_c * kp * kn, 0.0),
                             keepdims=True)

    x = x_ref[...]
    ones_1e = jnp.ones((1, E_), f32)

    def dist_row(j, _):
        row = x_ref[pl.ds(j, 1), :]
        diff = jnp.abs(x - row)
        d_row = lax.dot_general(ones_1e, diff, (((1,), (1,)), ((), ())),
                                precision=lax.Precision.HIGHEST)
        d_ref[pl.ds(j, 1), :] = d_row
        return 0

    lax.fori_loop(0, B_, dist_row, 0, unroll=2)


def _tc_stage(preds, target, x):
    return pl.pallas_call(
        _tc_body,
        out_shape=[
            jax.ShapeDtypeStruct((B_, B_), jnp.float32),
            jax.ShapeDtypeStruct((1, C_), jnp.float32),
            jax.ShapeDtypeStruct((1, 1), jnp.float32),
        ],
    )(preds, target, x)


def _lexmax(ka, ia, kb, ib):
    g = (ka > kb) | ((ka == kb) & (ia > ib))
    return jnp.where(g, ka, kb), jnp.where(g, ia, ib)


def _lexmin(ka, ia, kb, ib):
    g = (ka < kb) | ((ka == kb) & (ia < ib))
    return jnp.where(g, ka, kb), jnp.where(g, ia, ib)


def _sc_body(scores_hbm, target_hbm, d_hbm, valid_hbm, out_hbm,
             s_v, t_v, d_v, val_v, acc_v):
    f32, i32 = jnp.float32, jnp.int32
    wid = lax.axis_index("s") * 2 + lax.axis_index("c")
    iota = lax.broadcasted_iota(i32, (L_,), 0)

    pltpu.sync_copy(scores_hbm.at[pl.ds(wid * (CPW_ * B_), CPW_ * B_)], s_v)
    pltpu.sync_copy(target_hbm.at[pl.ds(wid * (CPW_ * B_), CPW_ * B_)], t_v)
    pltpu.sync_copy(d_hbm, d_v)
    pltpu.sync_copy(valid_hbm.at[pl.ds(wid * L_, L_)], val_v)

    acc_v[...] = jnp.zeros((L_,), f32)
    validv = val_v[...]

    def splat(x):
        return jnp.full((L_,), x, i32)

    def lane_bcast(vec, k):
        return jnp.take_along_axis(vec, splat(k), axis=0)

    def class_body(i, _):
        vsc = jnp.max(jnp.where(iota == i, validv, 0.0))

        @pl.when(vsc > 0.5)
        def _():
            base = i * B_
            sg = [s_v[pl.ds(base + g * L_, L_)] for g in range(4)]
            tg = [t_v[pl.ds(base + g * L_, L_)] for g in range(4)]
            mem = [tg[g] == 1 for g in range(4)]
            idxg = [iota + g * L_ for g in range(4)]
            pc = [plsc.all_reduce_population_count(mem[g]) for g in range(4)]
            n_spl = pc[0] + pc[1] + pc[2] + pc[3]
            kp_spl = jnp.minimum(n_spl - 1, K_)
            kn_spl = jnp.minimum(B_ - n_spl, K_)

            ninf = f32(-jnp.inf)

            def neg_step(k, carry):
                c0, c1, c2, c3, nidx = carry
                ka, ia = _lexmax(c0, idxg[0], c1, idxg[1])
                kb, ib = _lexmax(c2, idxg[2], c3, idxg[3])
                km, im = _lexmax(ka, ia, kb, ib)
                vmax = jnp.max(km)
                imax = jnp.max(jnp.where(km == vmax, im, -1))
                nidx = jnp.where(iota == k, imax, nidx)
                cs = [jnp.where(idxg[g] == imax, ninf, c)
                      for g, c in enumerate((c0, c1, c2, c3))]
                return cs[0], cs[1], cs[2], cs[3], nidx

            cur = [jnp.where(mem[g], ninf, sg[g]) for g in range(4)]
            *_, nidx = lax.fori_loop(
                0, K_, neg_step, (cur[0], cur[1], cur[2], cur[3], splat(0)))

            pinf = f32(jnp.inf)

            def pos_step(j, carry):
                c0, c1, c2, c3, pidx, p16 = carry
                ka, ia = _lexmin(c0, idxg[0], c1, idxg[1])
                kb, ib = _lexmin(c2, idxg[2], c3, idxg[3])
                km, im = _lexmin(ka, ia, kb, ib)
                vmin = jnp.min(km)
                imin = jnp.min(jnp.where(km == vmin, im, 99))
                imin = jnp.where(vmin < pinf, imin, 99)
                pidx = jnp.where((iota == j) & (j < K_), imin, pidx)
                p16 = jnp.where(j == K_, splat(imin), p16)
                cs = [jnp.where(idxg[g] == imin, pinf, c)
                      for g, c in enumerate((c0, c1, c2, c3))]
                return cs[0], cs[1], cs[2], cs[3], pidx, p16

            cur = [jnp.where(mem[g], sg[g], pinf) for g in range(4)]
            *_, pidx, p16 = lax.fori_loop(
                0, K_ + 1, pos_step,
                (cur[0], cur[1], cur[2], cur[3], splat(99), splat(99)))

            def chunk_body(g, _):
                off = base + g * L_
                mm = t_v[pl.ds(off, L_)] == 1
                npc = jnp.max(plsc.all_reduce_population_count(mm))

                @pl.when(npc > 0)
                def _():
                    b_vec = iota + g * L_
                    flat_b = b_vec * B_
                    dn = []
                    for k in range(K_):
                        col = lane_bcast(nidx, k)
                        dnk = plsc.load_gather(d_v, [flat_b + col])
                        dn.append(jnp.where(splat(k) < kn_spl, dnk, BIG_))

                    def entry_body(j, seen):
                        jc = jnp.minimum(j, K_ - 1)
                        pj = jnp.where(splat(j) < K_,
                                       jnp.take_along_axis(pidx, splat(jc),
                                                           axis=0),
                                       p16)
                        dpj = plsc.load_gather(
                            d_v, [flat_b + jnp.minimum(pj, B_ - 1)])
                        eq = pj == b_vec
                        validp = mm & (~eq) & ((splat(j) - seen) < kp_spl)
                        terms = jnp.zeros((L_,), f32)
                        for k in range(K_):
                            terms = terms + jnp.maximum(
                                dpj - dn[k] + MARGIN_, 0.0)
                        acc_v[...] = acc_v[...] + jnp.where(validp, terms, 0.0)
                        return seen + eq.astype(i32)

                    lax.fori_loop(0, K_ + 1, entry_body, splat(0))

                return 0

            lax.fori_loop(0, 4, chunk_body, 0)

        return 0

    lax.fori_loop(0, CPW_, class_body, 0)
    pltpu.sync_copy(acc_v, out_hbm.at[pl.ds(wid * L_, L_)])


@functools.cache
def _sc_stage():
    return pl.kernel(
        _sc_body,
        out_type=jax.ShapeDtypeStruct((NW_ * L_,), jnp.float32),
        mesh=plsc.VectorSubcoreMesh(
            core_axis_name="c", subcore_axis_name="s",
            num_cores=2, num_subcores=16),
        compiler_params=pltpu.CompilerParams(needs_layout_passes=False),
        scratch_types=[
            pltpu.VMEM((CPW_ * B_,), jnp.float32),
            pltpu.VMEM((CPW_ * B_,), jnp.int32),
            pltpu.VMEM((B_ * B_,), jnp.float32),
            pltpu.VMEM((L_,), jnp.float32),
            pltpu.VMEM((L_,), jnp.float32),
        ],
    )


@jax.jit
def kernel(input, target, X):
    dmat, valid, cnt = _tc_stage(input, target, X)
    scores_flat = input.T.reshape(-1)
    target_flat = target.T.astype(jnp.int32).reshape(-1)
    valid_pad = jnp.pad(valid.reshape(NW_, CPW_),
                        ((0, 0), (0, L_ - CPW_))).reshape(-1)
    partials = _sc_stage()(scores_flat, target_flat, dmat.reshape(-1),
                           valid_pad)
    total = jnp.sum(partials)
    count = cnt[0, 0]
    return jnp.where(count > 0.0, total / count, jnp.float32(0.0))


# speedup vs baseline: 2.0538x; 2.0538x over previous
import functools
import numpy as np
import jax
import jax.numpy as jnp
from jax import lax
from jax.experimental import pallas as pl
from jax.experimental.pallas import tpu as pltpu
from jax.experimental.pallas import tpu_sc as plsc

MARGIN_ = 0.5
C_ = 256
K_ = 16
B_ = 64
E_ = 4096
L_ = 16
NW_ = 32
CPW_ = C_ // NW_
BIG_ = np.float32(3.0e38)


def _tc_body(preds_ref, target_ref, x_ref,
             d_ref, st_ref, tt_ref, count_ref):
    f32 = jnp.float32
    t = target_ref[...]
    preds = preds_ref[...]

    iota_cc_r = lax.broadcasted_iota(jnp.int32, (C_, C_), 0)
    iota_cc_c = lax.broadcasted_iota(jnp.int32, (C_, C_), 1)
    eye_b = (lax.broadcasted_iota(jnp.int32, (B_, B_), 0) ==
             lax.broadcasted_iota(jnp.int32, (B_, B_), 1)).astype(f32)

    tf = t.astype(f32)
    h_row = jnp.sum(tf, axis=0, keepdims=True)
    ones_b1 = jnp.ones((B_, 1), f32)
    h_col = lax.dot_general(tf, ones_b1, (((0,), (0,)), ((), ())),
                            precision=lax.Precision.HIGHEST)

    before_t = (h_row < h_col) | ((h_row == h_col) & (iota_cc_c <= iota_cc_r))
    s_cum = jnp.sum(h_row * before_t.astype(f32), axis=1, keepdims=True)
    minority = (s_cum <= 0.5 * C_) & (h_col > 1.0)

    kp = jnp.minimum(h_col - 1.0, float(K_))
    kn = jnp.minimum(float(B_) - h_col, float(K_))
    class_valid = minority & (h_col < float(B_))
    count_ref[...] = jnp.sum(jnp.where(class_valid, h_col * kp * kn, 0.0),
                             keepdims=True)

    st_ref[...] = lax.dot_general(preds, eye_b, (((0,), (0,)), ((), ())),
                                  precision=lax.Precision.HIGHEST)
    tt = lax.dot_general(tf, eye_b, (((0,), (0,)), ((), ())),
                         precision=lax.Precision.HIGHEST)
    tt_ref[...] = (tt + 2.0 * class_valid.astype(f32)).astype(jnp.int32)

    xb = x_ref[...].astype(jnp.bfloat16)
    ones_1e = jnp.ones((1, E_), jnp.bfloat16)

    def dist_row(j, _):
        row = x_ref[pl.ds(j, 1), :].astype(jnp.bfloat16)
        diff = jnp.abs(xb - row)
        d_row = lax.dot_general(ones_1e, diff, (((1,), (1,)), ((), ())),
                                preferred_element_type=f32)
        d_ref[pl.ds(j, 1), :] = d_row
        return 0

    lax.fori_loop(0, B_, dist_row, 0, unroll=2)


def _tc_stage(preds, target, x):
    return pl.pallas_call(
        _tc_body,
        out_shape=[
            jax.ShapeDtypeStruct((B_, B_), jnp.float32),
            jax.ShapeDtypeStruct((C_, B_), jnp.float32),
            jax.ShapeDtypeStruct((C_, B_), jnp.int32),
            jax.ShapeDtypeStruct((1, 1), jnp.float32),
        ],
    )(preds, target, x)


def _lexmax(ka, ia, kb, ib):
    g = (ka > kb) | ((ka == kb) & (ia > ib))
    return jnp.where(g, ka, kb), jnp.where(g, ia, ib)


def _lexmin(ka, ia, kb, ib):
    g = (ka < kb) | ((ka == kb) & (ia < ib))
    return jnp.where(g, ka, kb), jnp.where(g, ia, ib)


def _sc_body(scores_hbm, target_hbm, d_hbm, out_hbm,
             s_v, t_v, d_v, acc_v):
    f32, i32 = jnp.float32, jnp.int32
    wid = lax.axis_index("s") * 2 + lax.axis_index("c")
    iota = lax.broadcasted_iota(i32, (L_,), 0)

    pltpu.sync_copy(scores_hbm.at[pl.ds(wid * CPW_, CPW_)], s_v)
    pltpu.sync_copy(target_hbm.at[pl.ds(wid * CPW_, CPW_)], t_v)
    pltpu.sync_copy(d_hbm, d_v)

    acc_v[...] = jnp.zeros((L_,), f32)

    def splat(x):
        return jnp.full((L_,), x, i32)

    def lane_bcast(vec, k):
        return jnp.take_along_axis(vec, splat(k), axis=0)

    def class_body(i, _):
        tg = [t_v[i, pl.ds(g * L_, L_)] for g in range(4)]
        vsc = jnp.max(tg[0])

        @pl.when(vsc >= 2)
        def _():
            sg = [s_v[i, pl.ds(g * L_, L_)] for g in range(4)]
            mem = [(tg[g] & 1) == 1 for g in range(4)]
            idxg = [iota + g * L_ for g in range(4)]
            pc = [plsc.all_reduce_population_count(mem[g]) for g in range(4)]
            n_spl = pc[0] + pc[1] + pc[2] + pc[3]
            kp_spl = jnp.minimum(n_spl - 1, K_)
            kn_spl = jnp.minimum(B_ - n_spl, K_)

            ninf = f32(-jnp.inf)

            def neg_step(k, carry):
                c0, c1, c2, c3, nidx = carry
                ka, ia = _lexmax(c0, idxg[0], c1, idxg[1])
                kb, ib = _lexmax(c2, idxg[2], c3, idxg[3])
                km, im = _lexmax(ka, ia, kb, ib)
                vmax = jnp.max(km)
                imax = jnp.max(jnp.where(km == vmax, im, -1))
                nidx = jnp.where(iota == k, imax, nidx)
                cs = [jnp.where(idxg[g] == imax, ninf, c)
                      for g, c in enumerate((c0, c1, c2, c3))]
                return cs[0], cs[1], cs[2], cs[3], nidx

            cur = [jnp.where(mem[g], ninf, sg[g]) for g in range(4)]
            *_, nidx = lax.fori_loop(
                0, K_, neg_step, (cur[0], cur[1], cur[2], cur[3], splat(0)))

            pinf = f32(jnp.inf)

            def pos_step(j, carry):
                c0, c1, c2, c3, pidx, p16 = carry
                ka, ia = _lexmin(c0, idxg[0], c1, idxg[1])
                kb, ib = _lexmin(c2, idxg[2], c3, idxg[3])
                km, im = _lexmin(ka, ia, kb, ib)
                vmin = jnp.min(km)
                imin = jnp.min(jnp.where(km == vmin, im, 99))
                imin = jnp.where(vmin < pinf, imin, 99)
                pidx = jnp.where((iota == j) & (j < K_), imin, pidx)
                p16 = jnp.where(j == K_, splat(imin), p16)
                cs = [jnp.where(idxg[g] == imin, pinf, c)
                      for g, c in enumerate((c0, c1, c2, c3))]
                return cs[0], cs[1], cs[2], cs[3], pidx, p16

            cur = [jnp.where(mem[g], sg[g], pinf) for g in range(4)]
            *_, pidx, p16 = lax.fori_loop(
                0, K_ + 1, pos_step,
                (cur[0], cur[1], cur[2], cur[3], splat(99), splat(99)))

            def chunk_body(g, _):
                mmv = t_v[i, pl.ds(g * L_, L_)]
                mm = (mmv & 1) == 1
                npc = jnp.max(plsc.all_reduce_population_count(mm))

                @pl.when(npc > 0)
                def _():
                    b_vec = iota + g * L_
                    dn = []
                    for k in range(K_):
                        col = lane_bcast(nidx, k)
                        dnk = plsc.load_gather(d_v, [b_vec, col])
                        dn.append(jnp.where(splat(k) < kn_spl, dnk, BIG_))

                    def entry_body(j, seen):
                        jc = jnp.minimum(j, K_ - 1)
                        pj = jnp.where(splat(j) < K_,
                                       jnp.take_along_axis(pidx, splat(jc),
                                                           axis=0),
                                       p16)
                        dpj = plsc.load_gather(
                            d_v, [b_vec, jnp.minimum(pj, B_ - 1)])
                        eq = pj == b_vec
                        validp = mm & (~eq) & ((splat(j) - seen) < kp_spl)
                        terms = jnp.zeros((L_,), f32)
                        for k in range(K_):
                            terms = terms + jnp.maximum(
                                dpj - dn[k] + MARGIN_, 0.0)
                        acc_v[...] = acc_v[...] + jnp.where(validp, terms, 0.0)
                        return seen + eq.astype(i32)

                    lax.fori_loop(0, K_ + 1, entry_body, splat(0))

                return 0

            lax.fori_loop(0, 4, chunk_body, 0)

        return 0

    lax.fori_loop(0, CPW_, class_body, 0)
    pltpu.sync_copy(acc_v, out_hbm.at[wid])


@functools.cache
def _sc_stage():
    return pl.kernel(
        _sc_body,
        out_type=jax.ShapeDtypeStruct((NW_, L_), jnp.float32),
        mesh=plsc.VectorSubcoreMesh(
            core_axis_name="c", subcore_axis_name="s",
            num_cores=2, num_subcores=16),
        compiler_params=pltpu.CompilerParams(needs_layout_passes=False),
        scratch_types=[
            pltpu.VMEM((CPW_, B_), jnp.float32),
            pltpu.VMEM((CPW_, B_), jnp.int32),
            pltpu.VMEM((B_, B_), jnp.float32),
            pltpu.VMEM((L_,), jnp.float32),
        ],
    )


@jax.jit
def kernel(input, target, X):
    dmat, scores_t, target_t, cnt = _tc_stage(input, target, X)
    partials = _sc_stage()(scores_t, target_t, dmat)
    count = cnt[0, 0]
    return jnp.where(count > 0.0, jnp.sum(partials) / count, jnp.float32(0.0))


# speedup vs baseline: 2.2292x; 1.0854x over previous
import functools
import numpy as np
import jax
import jax.numpy as jnp
from jax import lax
from jax.experimental import pallas as pl
from jax.experimental.pallas import tpu as pltpu
from jax.experimental.pallas import tpu_sc as plsc

MARGIN_ = 0.5
C_ = 256
K_ = 16
B_ = 64
E_ = 4096
L_ = 16
NW_ = 32
CPW_ = C_ // NW_
BIG_ = np.float32(3.0e38)


def _tc_body(preds_ref, target_ref, x_ref,
             d_ref, st_ref, tt_ref, count_ref):
    f32 = jnp.float32
    t = target_ref[...]
    preds = preds_ref[...]

    iota_cc_r = lax.broadcasted_iota(jnp.int32, (C_, C_), 0)
    iota_cc_c = lax.broadcasted_iota(jnp.int32, (C_, C_), 1)
    eye_b = (lax.broadcasted_iota(jnp.int32, (B_, B_), 0) ==
             lax.broadcasted_iota(jnp.int32, (B_, B_), 1)).astype(f32)

    tf = t.astype(f32)
    h_row = jnp.sum(tf, axis=0, keepdims=True)
    ones_b1 = jnp.ones((B_, 1), f32)
    h_col = lax.dot_general(tf, ones_b1, (((0,), (0,)), ((), ())))

    before_t = (h_row < h_col) | ((h_row == h_col) & (iota_cc_c <= iota_cc_r))
    s_cum = jnp.sum(h_row * before_t.astype(f32), axis=1, keepdims=True)
    minority = (s_cum <= 0.5 * C_) & (h_col > 1.0)

    kp = jnp.minimum(h_col - 1.0, float(K_))
    kn = jnp.minimum(float(B_) - h_col, float(K_))
    class_valid = minority & (h_col < float(B_))
    count_ref[...] = jnp.sum(jnp.where(class_valid, h_col * kp * kn, 0.0),
                             keepdims=True)

    st_ref[...] = lax.dot_general(preds, eye_b, (((0,), (0,)), ((), ())),
                                  precision=lax.Precision.HIGHEST)
    tt = lax.dot_general(tf, eye_b, (((0,), (0,)), ((), ())))
    tt_ref[...] = (tt + 2.0 * class_valid.astype(f32)).astype(jnp.int32)

    xb = x_ref[...].astype(jnp.bfloat16)
    ones_1e = jnp.ones((1, E_), jnp.bfloat16)

    def dist_row(j, _):
        row = x_ref[pl.ds(j, 1), :].astype(jnp.bfloat16)
        diff = jnp.abs(xb - row)
        d_row = lax.dot_general(ones_1e, diff, (((1,), (1,)), ((), ())),
                                preferred_element_type=f32)
        d_ref[pl.ds(j, 1), :] = d_row
        return 0

    lax.fori_loop(0, B_, dist_row, 0, unroll=8)


def _tc_stage(preds, target, x):
    return pl.pallas_call(
        _tc_body,
        out_shape=[
            jax.ShapeDtypeStruct((B_, B_), jnp.float32),
            jax.ShapeDtypeStruct((C_, B_), jnp.float32),
            jax.ShapeDtypeStruct((C_, B_), jnp.int32),
            jax.ShapeDtypeStruct((1, 1), jnp.float32),
        ],
    )(preds, target, x)


def _lexmax(ka, ia, kb, ib):
    g = (ka > kb) | ((ka == kb) & (ia > ib))
    return jnp.where(g, ka, kb), jnp.where(g, ia, ib)


def _lexmin(ka, ia, kb, ib):
    g = (ka < kb) | ((ka == kb) & (ia < ib))
    return jnp.where(g, ka, kb), jnp.where(g, ia, ib)


def _sc_body(scores_hbm, target_hbm, d_hbm, out_hbm,
             s_v, t_v, d_v, acc_v, sem):
    f32, i32 = jnp.float32, jnp.int32
    wid = lax.axis_index("s") * 2 + lax.axis_index("c")
    iota = lax.broadcasted_iota(i32, (L_,), 0)

    c1 = pltpu.async_copy(scores_hbm.at[pl.ds(wid * CPW_, CPW_)], s_v, sem)
    c2 = pltpu.async_copy(target_hbm.at[pl.ds(wid * CPW_, CPW_)], t_v, sem)
    c3 = pltpu.async_copy(d_hbm, d_v, sem)
    c1.wait()
    c2.wait()
    c3.wait()

    acc_v[...] = jnp.zeros((L_,), f32)

    def splat(x):
        return jnp.full((L_,), x, i32)

    def lane_bcast(vec, k):
        return jnp.take_along_axis(vec, splat(k), axis=0)

    def class_body(i, _):
        tg = [t_v[i, pl.ds(g * L_, L_)] for g in range(4)]
        vsc = jnp.max(tg[0])

        @pl.when(vsc >= 2)
        def _():
            sg = [s_v[i, pl.ds(g * L_, L_)] for g in range(4)]
            mem = [(tg[g] & 1) == 1 for g in range(4)]
            idxg = [iota + g * L_ for g in range(4)]
            pc = [plsc.all_reduce_population_count(mem[g]) for g in range(4)]
            n_spl = pc[0] + pc[1] + pc[2] + pc[3]
            kp_spl = jnp.minimum(n_spl - 1, K_)
            kn_spl = jnp.minimum(B_ - n_spl, K_)

            ninf = f32(-jnp.inf)

            def neg_step(k, carry):
                c0, c1, c2, c3, nidx = carry
                ka, ia = _lexmax(c0, idxg[0], c1, idxg[1])
                kb, ib = _lexmax(c2, idxg[2], c3, idxg[3])
                km, im = _lexmax(ka, ia, kb, ib)
                vmax = jnp.max(km)
                imax = jnp.max(jnp.where(km == vmax, im, -1))
                nidx = jnp.where(iota == k, imax, nidx)
                cs = [jnp.where(idxg[g] == imax, ninf, c)
                      for g, c in enumerate((c0, c1, c2, c3))]
                return cs[0], cs[1], cs[2], cs[3], nidx

            cur = [jnp.where(mem[g], ninf, sg[g]) for g in range(4)]
            *_, nidx = lax.fori_loop(
                0, K_, neg_step, (cur[0], cur[1], cur[2], cur[3], splat(0)))

            pinf = f32(jnp.inf)

            def pos_step(j, carry):
                c0, c1, c2, c3, pidx, p16 = carry
                ka, ia = _lexmin(c0, idxg[0], c1, idxg[1])
                kb, ib = _lexmin(c2, idxg[2], c3, idxg[3])
                km, im = _lexmin(ka, ia, kb, ib)
                vmin = jnp.min(km)
                imin = jnp.min(jnp.where(km == vmin, im, 99))
                imin = jnp.where(vmin < pinf, imin, 99)
                pidx = jnp.where((iota == j) & (j < K_), imin, pidx)
                p16 = jnp.where(j == K_, splat(imin), p16)
                cs = [jnp.where(idxg[g] == imin, pinf, c)
                      for g, c in enumerate((c0, c1, c2, c3))]
                return cs[0], cs[1], cs[2], cs[3], pidx, p16

            cur = [jnp.where(mem[g], sg[g], pinf) for g in range(4)]
            *_, pidx, p16 = lax.fori_loop(
                0, K_ + 1, pos_step,
                (cur[0], cur[1], cur[2], cur[3], splat(99), splat(99)))

            def chunk_body(g, _):
                mmv = t_v[i, pl.ds(g * L_, L_)]
                mm = (mmv & 1) == 1
                npc = jnp.max(plsc.all_reduce_population_count(mm))

                @pl.when(npc > 0)
                def _():
                    b_vec = iota + g * L_
                    dn = []
                    for k in range(K_):
                        col = lane_bcast(nidx, k)
                        dnk = plsc.load_gather(d_v, [b_vec, col])
                        dn.append(jnp.where(splat(k) < kn_spl, dnk, BIG_))

                    def entry_body(j, seen):
                        jc = jnp.minimum(j, K_ - 1)
                        pj = jnp.where(splat(j) < K_,
                                       jnp.take_along_axis(pidx, splat(jc),
                                                           axis=0),
                                       p16)
                        dpj = plsc.load_gather(
                            d_v, [b_vec, jnp.minimum(pj, B_ - 1)])
                        eq = pj == b_vec
                        validp = mm & (~eq) & ((splat(j) - seen) < kp_spl)
                        terms = jnp.zeros((L_,), f32)
                        for k in range(K_):
                            terms = terms + jnp.maximum(
                                dpj - dn[k] + MARGIN_, 0.0)
                        acc_v[...] = acc_v[...] + jnp.where(validp, terms, 0.0)
                        return seen + eq.astype(i32)

                    lax.fori_loop(0, K_ + 1, entry_body, splat(0))

                return 0

            lax.fori_loop(0, 4, chunk_body, 0)

        return 0

    lax.fori_loop(0, CPW_, class_body, 0)
    pltpu.sync_copy(acc_v, out_hbm.at[wid])


@functools.cache
def _sc_stage():
    return pl.kernel(
        _sc_body,
        out_type=jax.ShapeDtypeStruct((NW_, L_), jnp.float32),
        mesh=plsc.VectorSubcoreMesh(
            core_axis_name="c", subcore_axis_name="s",
            num_cores=2, num_subcores=16),
        compiler_params=pltpu.CompilerParams(needs_layout_passes=False),
        scratch_types=[
            pltpu.VMEM((CPW_, B_), jnp.float32),
            pltpu.VMEM((CPW_, B_), jnp.int32),
            pltpu.VMEM((B_, B_), jnp.float32),
            pltpu.VMEM((L_,), jnp.float32),
            pltpu.SemaphoreType.DMA,
        ],
    )


@jax.jit
def kernel(input, target, X):
    dmat, scores_t, target_t, cnt = _tc_stage(input, target, X)
    partials = _sc_stage()(scores_t, target_t, dmat)
    count = cnt[0, 0]
    return jnp.where(count > 0.0, jnp.sum(partials) / count, jnp.float32(0.0))
